# baseline (speedup 1.0000x reference)
import functools

import jax
import jax.numpy as jnp
from jax import lax
from jax.experimental import pallas as pl
from jax.experimental.pallas import tpu as pltpu
from jax.experimental.pallas import tpu_sc as plsc

_G = 64
_BN = 1000
_CH = 128
_NC = 2
_NS = 16
_NW = _NC * _NS
_NPAD = 10240
_DEGW = 8
_NBUF = 4


def _make_sc_agg(e_pad, width, nbuf):
    cpw = e_pad // (_NW * _CH)
    epw = cpw * _CH
    rpt = _NPAD // _NS
    assert cpw % nbuf == 0

    mesh = plsc.VectorSubcoreMesh(core_axis_name="c", subcore_axis_name="s")

    @functools.partial(
        pl.kernel,
        out_type=jax.ShapeDtypeStruct((_NC, _NPAD, width), jnp.float32),
        mesh=mesh,
        compiler_params=pltpu.CompilerParams(use_tc_tiling_on_sc=False),
        scratch_types=[
            pltpu.VMEM_SHARED((_NPAD, width), jnp.float32),
            pltpu.VMEM((nbuf, _CH), jnp.int32),
            pltpu.VMEM((nbuf, _CH), jnp.int32),
            pltpu.VMEM((nbuf, _CH, width), jnp.float32),
            pltpu.VMEM((16, width), jnp.float32),
        ] + [pltpu.SemaphoreType.DMA] * (2 * nbuf),
    )
    def agg(src_hbm, dst_hbm, g_hbm, out_hbm, acc_sh, src_v, dst_v, rows_v,
            zb_v, *sems):
        gsem = sems[:nbuf]
        ssem = sems[nbuf:]
        c = lax.axis_index("c")
        s = lax.axis_index("s")
        wid = c * _NS + s
        for i in range(16):
            for j in range(width // 16):
                zb_v[i, pl.ds(j * 16, 16)] = jnp.zeros((16,), jnp.float32)

        def zrow(i, carry):
            pltpu.sync_copy(zb_v, acc_sh.at[pl.ds(s * rpt + i * 16, 16)])
            return carry
        lax.fori_loop(0, rpt // 16, zrow, 0)
        plsc.subcore_barrier()

        base0 = wid * epw

        def load(k, b):
            base = base0 + k * _CH
            pltpu.sync_copy(src_hbm.at[pl.ds(base, _CH)], src_v.at[b])
            pltpu.sync_copy(dst_hbm.at[pl.ds(base, _CH)], dst_v.at[b])
            pltpu.async_copy(g_hbm.at[src_v.at[b]], rows_v.at[b], gsem[b])

        for b in range(nbuf):
            load(b, b)

        def group(gi, carry):
            k0 = gi * nbuf
            for b in range(nbuf):
                k = k0 + b
                nb = (b + 1) % nbuf
                @pl.when(jnp.logical_and(k + 1 >= nbuf, k + 1 < cpw))
                def _():
                    pltpu.make_async_copy(
                        rows_v.at[nb], acc_sh.at[dst_v.at[nb]],
                        ssem[nb]).wait()
                    load(k + 1, nb)
                pltpu.make_async_copy(
                    g_hbm.at[src_v.at[b]], rows_v.at[b], gsem[b]).wait()
                pltpu.async_copy(rows_v.at[b], acc_sh.at[dst_v.at[b]],
                                 ssem[b], add=True)
            return carry
        lax.fori_loop(0, cpw // nbuf, group, 0)
        for b in range(nbuf):
            pltpu.make_async_copy(rows_v.at[b], acc_sh.at[dst_v.at[b]],
                                  ssem[b]).wait()
        plsc.subcore_barrier()
        pltpu.sync_copy(acc_sh.at[pl.ds(s * rpt, rpt)],
                        out_hbm.at[c, pl.ds(s * rpt, rpt)])

    return agg


def _make_sc_deg(e_pad):
    cpw = e_pad // (_NW * _CH)
    epw = cpw * _CH
    rpt = _NPAD // _NS

    mesh = plsc.VectorSubcoreMesh(core_axis_name="c", subcore_axis_name="s")

    @functools.partial(
        pl.kernel,
        out_type=jax.ShapeDtypeStruct((_NC, _NPAD, _DEGW), jnp.float32),
        mesh=mesh,
        compiler_params=pltpu.CompilerParams(use_tc_tiling_on_sc=False),
        scratch_types=[
            pltpu.VMEM_SHARED((_NPAD, _DEGW), jnp.float32),
            pltpu.VMEM((2, _CH), jnp.int32),
            pltpu.VMEM((_CH, _DEGW), jnp.float32),
            pltpu.VMEM((16, _DEGW), jnp.float32),
        ] + [pltpu.SemaphoreType.DMA] * 2,
    )
    def deg(dst_hbm, out_hbm, acc_sh, dst_v, ones_v, zb_v, *ssem):
        c = lax.axis_index("c")
        s = lax.axis_index("s")
        wid = c * _NS + s
        for i in range(16):
            zb_v[i, pl.ds(0, 16)] = jnp.zeros((16,), jnp.float32)
        for i in range(_CH):
            ones_v[i, pl.ds(0, 16)] = jnp.ones((16,), jnp.float32)

        def zrow(i, carry):
            pltpu.sync_copy(zb_v, acc_sh.at[pl.ds(s * rpt + i * 16, 16)])
            return carry
        lax.fori_loop(0, rpt // 16, zrow, 0)
        plsc.subcore_barrier()

        base0 = wid * epw

        def load(k, b):
            pltpu.sync_copy(dst_hbm.at[pl.ds(base0 + k * _CH, _CH)],
                            dst_v.at[b])

        for b in range(2):
            load(b, b)

        def group(gi, carry):
            for b in range(2):
                k = 2 * gi + b
                nb = (b + 1) % 2
                @pl.when(jnp.logical_and(k + 1 >= 2, k + 1 < cpw))
                def _():
                    pltpu.make_async_copy(ones_v, acc_sh.at[dst_v.at[nb]],
                                          ssem[nb]).wait()
                    load(k + 1, nb)
                pltpu.async_copy(ones_v, acc_sh.at[dst_v.at[b]], ssem[b],
                                 add=True)
            return carry
        lax.fori_loop(0, cpw // 2, group, 0)
        for b in range(2):
            pltpu.make_async_copy(ones_v, acc_sh.at[dst_v.at[b]],
                                  ssem[b]).wait()
        plsc.subcore_barrier()
        pltpu.sync_copy(acc_sh.at[pl.ds(s * rpt, rpt)],
                        out_hbm.at[c, pl.ds(s * rpt, rpt)])

    return deg


def _tc_g1(x, W1, d0, d1):
    n, din = x.shape
    dout = W1.shape[1]

    def body(x_ref, w_ref, a_ref, b_ref, g1_ref, dis_ref):
        deg = a_ref[:, :1] + b_ref[:, :1] + 1.0
        dis = lax.rsqrt(deg)
        xw = jnp.dot(x_ref[...], w_ref[...], preferred_element_type=jnp.float32)
        g1_ref[...] = xw * dis
        dis_ref[...] = jnp.broadcast_to(dis, (_BN, 8))

    return pl.pallas_call(
        body,
        grid=(n // _BN,),
        in_specs=[
            pl.BlockSpec((_BN, din), lambda i: (i, 0)),
            pl.BlockSpec((din, dout), lambda i: (0, 0)),
            pl.BlockSpec((_BN, _DEGW), lambda i: (i, 0)),
            pl.BlockSpec((_BN, _DEGW), lambda i: (i, 0)),
        ],
        out_specs=[
            pl.BlockSpec((_BN, dout), lambda i: (i, 0)),
            pl.BlockSpec((_BN, 8), lambda i: (i, 0)),
        ],
        out_shape=[
            jax.ShapeDtypeStruct((n, dout), jnp.float32),
            jax.ShapeDtypeStruct((n, 8), jnp.float32),
        ],
    )(x, W1, d0, d1)


def _tc_h1(p0, p1, g1, dis, b1, W2):
    n, d1 = g1.shape
    d2 = W2.shape[1]

    def body(p0_ref, p1_ref, g1_ref, dis_ref, b1_ref, w2_ref, g2_ref):
        disc = dis_ref[:, :1]
        h1 = jnp.maximum(
            disc * (p0_ref[...] + p1_ref[...] + g1_ref[...]) + b1_ref[...], 0.0)
        g2_ref[...] = jnp.dot(h1, w2_ref[...],
                              preferred_element_type=jnp.float32) * disc

    return pl.pallas_call(
        body,
        grid=(n // _BN,),
        in_specs=[
            pl.BlockSpec((_BN, d1), lambda i: (i, 0)),
            pl.BlockSpec((_BN, d1), lambda i: (i, 0)),
            pl.BlockSpec((_BN, d1), lambda i: (i, 0)),
            pl.BlockSpec((_BN, 8), lambda i: (i, 0)),
            pl.BlockSpec((1, d1), lambda i: (0, 0)),
            pl.BlockSpec((d1, d2), lambda i: (0, 0)),
        ],
        out_specs=pl.BlockSpec((_BN, d2), lambda i: (i, 0)),
        out_shape=jax.ShapeDtypeStruct((n, d2), jnp.float32),
    )(p0, p1, g1, dis, b1, W2)


def _tc_final(q0, q1, g2, dis, b2, fc1_w, fc1_b, batch2d, fc2_w, fc2_b):
    n, d2 = g2.shape
    f1 = fc1_w.shape[1]
    nlab = fc2_w.shape[1]
    nblk = n // _BN

    def body(q0_ref, q1_ref, g2_ref, dis_ref, b2_ref, w1_ref, bb1_ref,
             bat_ref, w2_ref, bb2_ref, ge_ref, cnt_ref, ic_ref):
        j = pl.program_id(0)
        disc = dis_ref[:, :1]
        h2 = jnp.maximum(
            disc * (q0_ref[...] + q1_ref[...] + g2_ref[...]) + b2_ref[...], 0.0)
        a1 = jnp.dot(h2, w1_ref[...],
                     preferred_element_type=jnp.float32) + bb1_ref[...]
        ids = bat_ref[...]
        oh = (ids == lax.broadcasted_iota(jnp.int32, (_BN, _G), 1)
              ).astype(jnp.float32)
        sums = lax.dot_general(oh, a1, (((0,), (0,)), ((), ())),
                               preferred_element_type=jnp.float32)
        cnts = jnp.broadcast_to(jnp.sum(oh, axis=0)[:, None], (_G, 8))

        @pl.when(j == 0)
        def _():
            ge_ref[...] = sums
            cnt_ref[...] = cnts

        @pl.when(j > 0)
        def _():
            ge_ref[...] += sums
            cnt_ref[...] += cnts

        @pl.when(j == nblk - 1)
        def _():
            ge = ge_ref[...] / jnp.maximum(cnt_ref[:, :1], 1.0)
            ge_ref[...] = ge
            ic_ref[...] = jnp.dot(ge, w2_ref[...],
                                  preferred_element_type=jnp.float32) + bb2_ref[...]

    return pl.pallas_call(
        body,
        grid=(nblk,),
        in_specs=[
            pl.BlockSpec((_BN, d2), lambda i: (i, 0)),
            pl.BlockSpec((_BN, d2), lambda i: (i, 0)),
            pl.BlockSpec((_BN, d2), lambda i: (i, 0)),
            pl.BlockSpec((_BN, 8), lambda i: (i, 0)),
            pl.BlockSpec((1, d2), lambda i: (0, 0)),
            pl.BlockSpec((d2, f1), lambda i: (0, 0)),
            pl.BlockSpec((1, f1), lambda i: (0, 0)),
            pl.BlockSpec((_BN, 1), lambda i: (i, 0)),
            pl.BlockSpec((f1, nlab), lambda i: (0, 0)),
            pl.BlockSpec((1, nlab), lambda i: (0, 0)),
        ],
        out_specs=[
            pl.BlockSpec((_G, f1), lambda i: (0, 0)),
            pl.BlockSpec((_G, 8), lambda i: (0, 0)),
            pl.BlockSpec((_G, nlab), lambda i: (0, 0)),
        ],
        out_shape=[
            jax.ShapeDtypeStruct((_G, f1), jnp.float32),
            jax.ShapeDtypeStruct((_G, 8), jnp.float32),
            jax.ShapeDtypeStruct((_G, nlab), jnp.float32),
        ],
    )(q0, q1, g2, dis, b2, fc1_w, fc1_b, batch2d, fc2_w, fc2_b)


def kernel(x, edge_index, batch, W1, b1, W2, b2, fc1_w, fc1_b, fc2_w, fc2_b):
    n = x.shape[0]
    e = edge_index.shape[1]

    grain = _NW * _CH * _NBUF
    e_pad = -(-e // grain) * grain
    pad = e_pad - e
    src = jnp.concatenate([edge_index[0], jnp.zeros((pad,), jnp.int32)])
    dst = jnp.concatenate([edge_index[1], jnp.full((pad,), n, jnp.int32)])

    degp = _make_sc_deg(e_pad)(dst)
    g1, dis = _tc_g1(x, W1, degp[0, :n], degp[1, :n])

    p = _make_sc_agg(e_pad, W1.shape[1], 2)(src, dst, g1)
    g2 = _tc_h1(p[0, :n], p[1, :n], g1, dis, b1.reshape(1, -1), W2)

    q = _make_sc_agg(e_pad, W2.shape[1], 4)(src, dst, g2)
    ge, _, ic = _tc_final(q[0, :n], q[1, :n], g2, dis, b2.reshape(1, -1),
                          fc1_w, fc1_b.reshape(1, -1), batch.reshape(-1, 1),
                          fc2_w, fc2_b.reshape(1, -1))
    return (ge, jnp.float32(0.0), ic)


# speedup vs baseline: 1.0445x; 1.0445x over previous
import functools

import jax
import jax.numpy as jnp
from jax import lax
from jax.experimental import pallas as pl
from jax.experimental.pallas import tpu as pltpu
from jax.experimental.pallas import tpu_sc as plsc

_G = 64
_BN = 1000
_CH = 128
_NC = 2
_NS = 16
_NW = _NC * _NS
_NPAD = 10240
_DEGW = 16
_NBUF = 4


def _make_sc_agg(e_pad, width, nbuf):
    cpw = e_pad // (_NW * _CH)
    epw = cpw * _CH
    rpt = _NPAD // _NS
    assert cpw % nbuf == 0

    mesh = plsc.VectorSubcoreMesh(core_axis_name="c", subcore_axis_name="s")

    @functools.partial(
        pl.kernel,
        out_type=jax.ShapeDtypeStruct((_NC, _NPAD, width), jnp.float32),
        mesh=mesh,
        compiler_params=pltpu.CompilerParams(use_tc_tiling_on_sc=False),
        scratch_types=[
            pltpu.VMEM_SHARED((_NPAD, width), jnp.float32),
            pltpu.VMEM((nbuf, _CH), jnp.int32),
            pltpu.VMEM((nbuf, _CH), jnp.int32),
            pltpu.VMEM((nbuf, _CH, width), jnp.float32),
            pltpu.VMEM((16, width), jnp.float32),
        ] + [pltpu.SemaphoreType.DMA] * nbuf,
    )
    def agg(src_hbm, dst_hbm, g_hbm, out_hbm, acc_sh, src_v, dst_v, rows_v,
            zb_v, *sems):
        gsem = sems
        c = lax.axis_index("c")
        s = lax.axis_index("s")
        wid = c * _NS + s
        for i in range(16):
            for j in range(width // 16):
                zb_v[i, pl.ds(j * 16, 16)] = jnp.zeros((16,), jnp.float32)

        def zrow(i, carry):
            pltpu.sync_copy(zb_v, acc_sh.at[pl.ds(s * rpt + i * 16, 16)])
            return carry
        lax.fori_loop(0, rpt // 16, zrow, 0)
        plsc.subcore_barrier()

        base0 = wid * epw

        def load(k, b):
            base = base0 + k * _CH
            pltpu.sync_copy(src_hbm.at[pl.ds(base, _CH)], src_v.at[b])
            pltpu.sync_copy(dst_hbm.at[pl.ds(base, _CH)], dst_v.at[b])
            pltpu.async_copy(g_hbm.at[src_v.at[b]], rows_v.at[b], gsem[b])

        for b in range(nbuf):
            load(b, b)

        def group(gi, carry):
            k0 = gi * nbuf
            for b in range(nbuf):
                k = k0 + b
                pltpu.make_async_copy(
                    g_hbm.at[src_v.at[b]], rows_v.at[b], gsem[b]).wait()
                pltpu.sync_copy(rows_v.at[b], acc_sh.at[dst_v.at[b]],
                                add=True)
                @pl.when(k + nbuf < cpw)
                def _():
                    load(k + nbuf, b)
            return carry
        lax.fori_loop(0, cpw // nbuf, group, 0)
        plsc.subcore_barrier()
        pltpu.sync_copy(acc_sh.at[pl.ds(s * rpt, rpt)],
                        out_hbm.at[c, pl.ds(s * rpt, rpt)])

    return agg


def _make_sc_deg(e_pad):
    cpw = e_pad // (_NW * _CH)
    epw = cpw * _CH
    rpt = _NPAD // _NS

    mesh = plsc.VectorSubcoreMesh(core_axis_name="c", subcore_axis_name="s")

    @functools.partial(
        pl.kernel,
        out_type=jax.ShapeDtypeStruct((_NC, _NPAD, _DEGW), jnp.float32),
        mesh=mesh,
        compiler_params=pltpu.CompilerParams(use_tc_tiling_on_sc=False),
        scratch_types=[
            pltpu.VMEM_SHARED((_NPAD, _DEGW), jnp.float32),
            pltpu.VMEM((2, _CH), jnp.int32),
            pltpu.VMEM((_CH, _DEGW), jnp.float32),
            pltpu.VMEM((16, _DEGW), jnp.float32),
        ],
    )
    def deg(dst_hbm, out_hbm, acc_sh, dst_v, ones_v, zb_v):
        c = lax.axis_index("c")
        s = lax.axis_index("s")
        wid = c * _NS + s
        for i in range(16):
            zb_v[i, pl.ds(0, 16)] = jnp.zeros((16,), jnp.float32)
        for i in range(_CH):
            ones_v[i, pl.ds(0, 16)] = jnp.ones((16,), jnp.float32)

        def zrow(i, carry):
            pltpu.sync_copy(zb_v, acc_sh.at[pl.ds(s * rpt + i * 16, 16)])
            return carry
        lax.fori_loop(0, rpt // 16, zrow, 0)
        plsc.subcore_barrier()

        base0 = wid * epw

        def chunk(k, carry):
            pltpu.sync_copy(dst_hbm.at[pl.ds(base0 + k * _CH, _CH)],
                            dst_v.at[0])
            pltpu.sync_copy(ones_v, acc_sh.at[dst_v.at[0]], add=True)
            return carry
        lax.fori_loop(0, cpw, chunk, 0)
        plsc.subcore_barrier()
        pltpu.sync_copy(acc_sh.at[pl.ds(s * rpt, rpt)],
                        out_hbm.at[c, pl.ds(s * rpt, rpt)])

    return deg


def _tc_g1(x, W1, d0, d1):
    n, din = x.shape
    dout = W1.shape[1]

    def body(x_ref, w_ref, a_ref, b_ref, g1_ref, dis_ref):
        deg = a_ref[:, :1] + b_ref[:, :1] + 1.0
        dis = 1.0 / jnp.sqrt(deg)
        xw = jnp.dot(x_ref[...], w_ref[...], preferred_element_type=jnp.float32)
        g1_ref[...] = xw * dis
        dis_ref[...] = jnp.broadcast_to(dis, (_BN, 8))

    return pl.pallas_call(
        body,
        grid=(n // _BN,),
        in_specs=[
            pl.BlockSpec((_BN, din), lambda i: (i, 0)),
            pl.BlockSpec((din, dout), lambda i: (0, 0)),
            pl.BlockSpec((_BN, _DEGW), lambda i: (i, 0)),
            pl.BlockSpec((_BN, _DEGW), lambda i: (i, 0)),
        ],
        out_specs=[
            pl.BlockSpec((_BN, dout), lambda i: (i, 0)),
            pl.BlockSpec((_BN, 8), lambda i: (i, 0)),
        ],
        out_shape=[
            jax.ShapeDtypeStruct((n, dout), jnp.float32),
            jax.ShapeDtypeStruct((n, 8), jnp.float32),
        ],
    )(x, W1, d0, d1)


def _tc_h1(p0, p1, g1, dis, b1, W2):
    n, d1 = g1.shape
    d2 = W2.shape[1]

    def body(p0_ref, p1_ref, g1_ref, dis_ref, b1_ref, w2_ref, g2_ref):
        disc = dis_ref[:, :1]
        h1 = jnp.maximum(
            disc * (p0_ref[...] + p1_ref[...] + g1_ref[...]) + b1_ref[...], 0.0)
        g2_ref[...] = jnp.dot(h1, w2_ref[...],
                              preferred_element_type=jnp.float32) * disc

    return pl.pallas_call(
        body,
        grid=(n // _BN,),
        in_specs=[
            pl.BlockSpec((_BN, d1), lambda i: (i, 0)),
            pl.BlockSpec((_BN, d1), lambda i: (i, 0)),
            pl.BlockSpec((_BN, d1), lambda i: (i, 0)),
            pl.BlockSpec((_BN, 8), lambda i: (i, 0)),
            pl.BlockSpec((1, d1), lambda i: (0, 0)),
            pl.BlockSpec((d1, d2), lambda i: (0, 0)),
        ],
        out_specs=pl.BlockSpec((_BN, d2), lambda i: (i, 0)),
        out_shape=jax.ShapeDtypeStruct((n, d2), jnp.float32),
    )(p0, p1, g1, dis, b1, W2)


def _tc_final(q0, q1, g2, dis, b2, fc1_w, fc1_b, batch2d, fc2_w, fc2_b):
    n, d2 = g2.shape
    f1 = fc1_w.shape[1]
    nlab = fc2_w.shape[1]
    nblk = n // _BN

    def body(q0_ref, q1_ref, g2_ref, dis_ref, b2_ref, w1_ref, bb1_ref,
             bat_ref, w2_ref, bb2_ref, ge_ref, cnt_ref, ic_ref):
        j = pl.program_id(0)
        disc = dis_ref[:, :1]
        h2 = jnp.maximum(
            disc * (q0_ref[...] + q1_ref[...] + g2_ref[...]) + b2_ref[...], 0.0)
        a1 = jnp.dot(h2, w1_ref[...],
                     preferred_element_type=jnp.float32) + bb1_ref[...]
        ids = bat_ref[...]
        oh = (ids == lax.broadcasted_iota(jnp.int32, (_BN, _G), 1)
              ).astype(jnp.float32)
        sums = lax.dot_general(oh, a1, (((0,), (0,)), ((), ())),
                               preferred_element_type=jnp.float32)
        cnts = jnp.broadcast_to(jnp.sum(oh, axis=0)[:, None], (_G, 8))

        @pl.when(j == 0)
        def _():
            ge_ref[...] = sums
            cnt_ref[...] = cnts

        @pl.when(j > 0)
        def _():
            ge_ref[...] += sums
            cnt_ref[...] += cnts

        @pl.when(j == nblk - 1)
        def _():
            ge = ge_ref[...] / jnp.maximum(cnt_ref[:, :1], 1.0)
            ge_ref[...] = ge
            ic_ref[...] = jnp.dot(ge, w2_ref[...],
                                  preferred_element_type=jnp.float32) + bb2_ref[...]

    return pl.pallas_call(
        body,
        grid=(nblk,),
        in_specs=[
            pl.BlockSpec((_BN, d2), lambda i: (i, 0)),
            pl.BlockSpec((_BN, d2), lambda i: (i, 0)),
            pl.BlockSpec((_BN, d2), lambda i: (i, 0)),
            pl.BlockSpec((_BN, 8), lambda i: (i, 0)),
            pl.BlockSpec((1, d2), lambda i: (0, 0)),
            pl.BlockSpec((d2, f1), lambda i: (0, 0)),
            pl.BlockSpec((1, f1), lambda i: (0, 0)),
            pl.BlockSpec((_BN, 1), lambda i: (i, 0)),
            pl.BlockSpec((f1, nlab), lambda i: (0, 0)),
            pl.BlockSpec((1, nlab), lambda i: (0, 0)),
        ],
        out_specs=[
            pl.BlockSpec((_G, f1), lambda i: (0, 0)),
            pl.BlockSpec((_G, 8), lambda i: (0, 0)),
            pl.BlockSpec((_G, nlab), lambda i: (0, 0)),
        ],
        out_shape=[
            jax.ShapeDtypeStruct((_G, f1), jnp.float32),
            jax.ShapeDtypeStruct((_G, 8), jnp.float32),
            jax.ShapeDtypeStruct((_G, nlab), jnp.float32),
        ],
    )(q0, q1, g2, dis, b2, fc1_w, fc1_b, batch2d, fc2_w, fc2_b)


def kernel(x, edge_index, batch, W1, b1, W2, b2, fc1_w, fc1_b, fc2_w, fc2_b):
    n = x.shape[0]
    e = edge_index.shape[1]

    grain = _NW * _CH * _NBUF
    e_pad = -(-e // grain) * grain
    pad = e_pad - e
    src = jnp.concatenate([edge_index[0], jnp.zeros((pad,), jnp.int32)])
    dst = jnp.concatenate([edge_index[1], jnp.full((pad,), n, jnp.int32)])

    degp = _make_sc_deg(e_pad)(dst)
    g1, dis = _tc_g1(x, W1, degp[0, :n], degp[1, :n])

    p = _make_sc_agg(e_pad, W1.shape[1], 2)(src, dst, g1)
    g2 = _tc_h1(p[0, :n], p[1, :n], g1, dis, b1.reshape(1, -1), W2)

    q = _make_sc_agg(e_pad, W2.shape[1], 4)(src, dst, g2)
    ge, _, ic = _tc_final(q[0, :n], q[1, :n], g2, dis, b2.reshape(1, -1),
                          fc1_w, fc1_b.reshape(1, -1), batch.reshape(-1, 1),
                          fc2_w, fc2_b.reshape(1, -1))
    return (ge, jnp.float32(0.0), ic)


# speedup vs baseline: 1.5916x; 1.5238x over previous
import functools

import jax
import jax.numpy as jnp
from jax import lax
from jax.experimental import pallas as pl
from jax.experimental.pallas import tpu as pltpu
from jax.experimental.pallas import tpu_sc as plsc

_G = 64
_BN = 1000
_CH = 128
_NC = 2
_NS = 16
_NW = _NC * _NS
_NPAD = 10240
_DEGW = 16
_NBUF = 4
_DELTA = 2


def _make_sc_agg(e_pad, width):
    cpt = e_pad // (_NS * _CH)
    rpt = _NPAD // _NS
    assert cpt % _NBUF == 0

    mesh = plsc.VectorSubcoreMesh(core_axis_name="c", subcore_axis_name="s")

    @functools.partial(
        pl.kernel,
        out_type=jax.ShapeDtypeStruct((_NC, _NPAD, width), jnp.float32),
        mesh=mesh,
        compiler_params=pltpu.CompilerParams(use_tc_tiling_on_sc=False),
        scratch_types=[
            pltpu.VMEM_SHARED((_NPAD, width), jnp.float32),
            pltpu.VMEM((cpt, _CH), jnp.int32),
            pltpu.VMEM((cpt, _CH), jnp.int32),
            pltpu.VMEM((_NBUF, _CH, width), jnp.float32),
            pltpu.VMEM((16, width), jnp.float32),
        ] + [pltpu.SemaphoreType.DMA] * (2 * _NBUF),
    )
    def agg(src_hbm, dst_hbm, g_hbm, out_hbm, acc_sh, src_v, dst_v, rows_v,
            zb_v, *sems):
        gsem = sems[:_NBUF]
        ssem = sems[_NBUF:]
        c = lax.axis_index("c")
        s = lax.axis_index("s")
        for i in range(16):
            for j in range(width // 16):
                zb_v[i, pl.ds(j * 16, 16)] = jnp.zeros((16,), jnp.float32)

        def zrow(i, carry):
            pltpu.sync_copy(zb_v, acc_sh.at[pl.ds(s * rpt + i * 16, 16)])
            return carry
        lax.fori_loop(0, rpt // 16, zrow, 0)

        pltpu.sync_copy(src_hbm.at[c, pl.ds(s * cpt, cpt)], src_v)
        pltpu.sync_copy(dst_hbm.at[pl.ds(s * cpt, cpt)], dst_v)
        plsc.subcore_barrier()

        def gather(k, b):
            pltpu.async_copy(g_hbm.at[src_v.at[k]], rows_v.at[b], gsem[b])

        def gather_wait(b):
            pltpu.make_async_copy(g_hbm.at[src_v.at[0]], rows_v.at[b],
                                  gsem[b]).wait()

        def scatter(k, b):
            pltpu.async_copy(rows_v.at[b], acc_sh.at[dst_v.at[k]], ssem[b],
                             add=True)

        def scatter_wait(b):
            pltpu.make_async_copy(rows_v.at[b], acc_sh.at[dst_v.at[0]],
                                  ssem[b]).wait()

        for b in range(_NBUF):
            gather(b, b)

        def group(gi, carry):
            k0 = gi * _NBUF
            for b in range(_NBUF):
                k = k0 + b
                gather_wait(b)
                scatter(k, b)
                bb = (b - _DELTA) % _NBUF
                kk = k - _DELTA + _NBUF

                @pl.when(jnp.logical_and(k >= _DELTA, kk < cpt))
                def _():
                    scatter_wait(bb)
                    gather(kk, bb)
            return carry
        lax.fori_loop(0, cpt // _NBUF, group, 0)
        for b in range(_NBUF):
            scatter_wait(b)
        plsc.subcore_barrier()
        pltpu.sync_copy(acc_sh.at[pl.ds(s * rpt, rpt)],
                        out_hbm.at[c, pl.ds(s * rpt, rpt)])

    return agg


def _make_sc_deg(e_pad):
    cpt = e_pad // (_NW * _CH)
    rpt = _NPAD // _NS
    nsem = 4

    mesh = plsc.VectorSubcoreMesh(core_axis_name="c", subcore_axis_name="s")

    @functools.partial(
        pl.kernel,
        out_type=jax.ShapeDtypeStruct((_NC, _NPAD, _DEGW), jnp.float32),
        mesh=mesh,
        compiler_params=pltpu.CompilerParams(use_tc_tiling_on_sc=False),
        scratch_types=[
            pltpu.VMEM_SHARED((_NPAD, _DEGW), jnp.float32),
            pltpu.VMEM((cpt, _CH), jnp.int32),
            pltpu.VMEM((_CH, _DEGW), jnp.float32),
            pltpu.VMEM((16, _DEGW), jnp.float32),
        ] + [pltpu.SemaphoreType.DMA] * nsem,
    )
    def deg(dst_hbm, out_hbm, acc_sh, dst_v, ones_v, zb_v, *ssem):
        c = lax.axis_index("c")
        s = lax.axis_index("s")
        wid = c * _NS + s
        for i in range(16):
            zb_v[i, pl.ds(0, 16)] = jnp.zeros((16,), jnp.float32)
        for i in range(_CH):
            ones_v[i, pl.ds(0, 16)] = jnp.ones((16,), jnp.float32)

        def zrow(i, carry):
            pltpu.sync_copy(zb_v, acc_sh.at[pl.ds(s * rpt + i * 16, 16)])
            return carry
        lax.fori_loop(0, rpt // 16, zrow, 0)
        pltpu.sync_copy(dst_hbm.at[pl.ds(wid * cpt, cpt)], dst_v)
        plsc.subcore_barrier()

        def chunk(k, carry):
            pltpu.sync_copy(ones_v, acc_sh.at[dst_v.at[k]], add=True)
            return carry
        lax.fori_loop(0, cpt, chunk, 0)
        plsc.subcore_barrier()
        pltpu.sync_copy(acc_sh.at[pl.ds(s * rpt, rpt)],
                        out_hbm.at[c, pl.ds(s * rpt, rpt)])

    return deg


def _tc_g1(x, W1, d0, d1):
    n, din = x.shape
    dout = W1.shape[1]
    half = dout // 2
    nb = n // _BN

    def body(x_ref, w_ref, a_ref, b_ref, g1_ref, dis_ref):
        deg = a_ref[:, :1] + b_ref[:, :1] + 1.0
        dis = 1.0 / jnp.sqrt(deg)
        xw = jnp.dot(x_ref[0], w_ref[0],
                     preferred_element_type=jnp.float32)
        g1_ref[...] = xw * dis
        dis_ref[...] = jnp.broadcast_to(dis, (_BN, 8))

    return pl.pallas_call(
        body,
        grid=(2 * nb,),
        in_specs=[
            pl.BlockSpec((1, _BN, din), lambda i: (0, i % nb, 0)),
            pl.BlockSpec((1, din, half), lambda i: (i // nb, 0, 0)),
            pl.BlockSpec((_BN, _DEGW), lambda i: (i % nb, 0)),
            pl.BlockSpec((_BN, _DEGW), lambda i: (i % nb, 0)),
        ],
        out_specs=[
            pl.BlockSpec((_BN, half), lambda i: (i, 0)),
            pl.BlockSpec((_BN, 8), lambda i: (i % nb, 0)),
        ],
        out_shape=[
            jax.ShapeDtypeStruct((2 * n, half), jnp.float32),
            jax.ShapeDtypeStruct((n, 8), jnp.float32),
        ],
    )(x.reshape(1, n, din),
      jnp.stack([W1[:, :half], W1[:, half:]]), d0, d1)


def _tc_h1(pa, pb, g1s, dis, b1, W2):
    n = pa.shape[0]
    d1 = 2 * pa.shape[1]
    d2 = W2.shape[1]
    half = d2 // 2
    nb = n // _BN

    def body(pa_ref, pb_ref, ga_ref, gb_ref, dis_ref, b1_ref, w2_ref, g2_ref):
        disc = dis_ref[:, :1]
        agg = jnp.concatenate([pa_ref[...], pb_ref[...]], axis=1)
        g1 = jnp.concatenate([ga_ref[...], gb_ref[...]], axis=1)
        h1 = jnp.maximum(disc * (agg + g1) + b1_ref[...], 0.0)
        g2_ref[...] = jnp.dot(h1, w2_ref[0],
                              preferred_element_type=jnp.float32) * disc

    return pl.pallas_call(
        body,
        grid=(2 * nb,),
        in_specs=[
            pl.BlockSpec((_BN, d1 // 2), lambda i: (i % nb, 0)),
            pl.BlockSpec((_BN, d1 // 2), lambda i: (i % nb, 0)),
            pl.BlockSpec((_BN, d1 // 2), lambda i: (i % nb, 0)),
            pl.BlockSpec((_BN, d1 // 2), lambda i: (nb + i % nb, 0)),
            pl.BlockSpec((_BN, 8), lambda i: (i % nb, 0)),
            pl.BlockSpec((1, d1), lambda i: (0, 0)),
            pl.BlockSpec((1, d1, half), lambda i: (i // nb, 0, 0)),
        ],
        out_specs=pl.BlockSpec((_BN, half), lambda i: (i, 0)),
        out_shape=jax.ShapeDtypeStruct((2 * n, half), jnp.float32),
    )(pa, pb, g1s, g1s, dis, b1,
      jnp.stack([W2[:, :half], W2[:, half:]]))


def _tc_final(qa, qb, g2s, dis, b2, fc1_w, fc1_b, batch2d, fc2_w, fc2_b):
    n = qa.shape[0]
    d2 = 2 * qa.shape[1]
    f1 = fc1_w.shape[1]
    nlab = fc2_w.shape[1]
    nb = n // _BN

    def body(qa_ref, qb_ref, ga_ref, gb_ref, dis_ref, b2_ref, w1_ref, bb1_ref,
             bat_ref, w2_ref, bb2_ref, ge_ref, cnt_ref, ic_ref):
        j = pl.program_id(0)
        disc = dis_ref[:, :1]
        agg = jnp.concatenate([qa_ref[...], qb_ref[...]], axis=1)
        g2 = jnp.concatenate([ga_ref[...], gb_ref[...]], axis=1)
        h2 = jnp.maximum(disc * (agg + g2) + b2_ref[...], 0.0)
        a1 = jnp.dot(h2, w1_ref[...],
                     preferred_element_type=jnp.float32) + bb1_ref[...]
        ids = bat_ref[...]
        oh = (ids == lax.broadcasted_iota(jnp.int32, (_BN, _G), 1)
              ).astype(jnp.float32)
        sums = lax.dot_general(oh, a1, (((0,), (0,)), ((), ())),
                               preferred_element_type=jnp.float32)
        cnts = jnp.broadcast_to(jnp.sum(oh, axis=0)[:, None], (_G, 8))

        @pl.when(j == 0)
        def _():
            ge_ref[...] = sums
            cnt_ref[...] = cnts

        @pl.when(j > 0)
        def _():
            ge_ref[...] += sums
            cnt_ref[...] += cnts

        @pl.when(j == nb - 1)
        def _():
            ge = ge_ref[...] / jnp.maximum(cnt_ref[:, :1], 1.0)
            ge_ref[...] = ge
            ic_ref[...] = jnp.dot(ge, w2_ref[...],
                                  preferred_element_type=jnp.float32) + bb2_ref[...]

    return pl.pallas_call(
        body,
        grid=(nb,),
        in_specs=[
            pl.BlockSpec((_BN, d2 // 2), lambda i: (i, 0)),
            pl.BlockSpec((_BN, d2 // 2), lambda i: (i, 0)),
            pl.BlockSpec((_BN, d2 // 2), lambda i: (i, 0)),
            pl.BlockSpec((_BN, d2 // 2), lambda i: (nb + i, 0)),
            pl.BlockSpec((_BN, 8), lambda i: (i, 0)),
            pl.BlockSpec((1, d2), lambda i: (0, 0)),
            pl.BlockSpec((d2, f1), lambda i: (0, 0)),
            pl.BlockSpec((1, f1), lambda i: (0, 0)),
            pl.BlockSpec((_BN, 1), lambda i: (i, 0)),
            pl.BlockSpec((f1, nlab), lambda i: (0, 0)),
            pl.BlockSpec((1, nlab), lambda i: (0, 0)),
        ],
        out_specs=[
            pl.BlockSpec((_G, f1), lambda i: (0, 0)),
            pl.BlockSpec((_G, 8), lambda i: (0, 0)),
            pl.BlockSpec((_G, nlab), lambda i: (0, 0)),
        ],
        out_shape=[
            jax.ShapeDtypeStruct((_G, f1), jnp.float32),
            jax.ShapeDtypeStruct((_G, 8), jnp.float32),
            jax.ShapeDtypeStruct((_G, nlab), jnp.float32),
        ],
    )(qa, qb, g2s, g2s, dis, b2, fc1_w, fc1_b, batch2d, fc2_w, fc2_b)


def kernel(x, edge_index, batch, W1, b1, W2, b2, fc1_w, fc1_b, fc2_w, fc2_b):
    n = x.shape[0]
    e = edge_index.shape[1]

    grain = _NW * _CH * _NBUF
    e_pad = -(-e // grain) * grain
    pad = e_pad - e
    src = jnp.concatenate([edge_index[0], jnp.zeros((pad,), jnp.int32)])
    dst = jnp.concatenate([edge_index[1], jnp.full((pad,), n, jnp.int32)])
    src2 = jnp.stack([src, src + n]).reshape(_NC, -1, _CH)
    dst3 = dst.reshape(-1, _CH)

    degp = _make_sc_deg(e_pad)(dst3)
    g1s, dis = _tc_g1(x, W1, degp[0, :n], degp[1, :n])

    p = _make_sc_agg(e_pad, W1.shape[1] // 2)(src2, dst3, g1s)
    g2s = _tc_h1(p[0, :n], p[1, :n], g1s, dis, b1.reshape(1, -1), W2)

    q = _make_sc_agg(e_pad, W2.shape[1] // 2)(src2, dst3, g2s)
    ge, _, ic = _tc_final(q[0, :n], q[1, :n], g2s, dis, b2.reshape(1, -1),
                          fc1_w, fc1_b.reshape(1, -1), batch.reshape(-1, 1),
                          fc2_w, fc2_b.reshape(1, -1))
    return (ge, jnp.float32(0.0), ic)
